# baseline (speedup 1.0000x reference)
import functools

import jax
import jax.numpy as jnp
from jax import lax
from jax.experimental import pallas as pl
from jax.experimental.pallas import tpu as pltpu
from jax.experimental.pallas import tpu_sc as plsc

_BATCH = 4096
_HIST = 50
_D = 64


@functools.lru_cache(maxsize=None)
def _build():
    info = plsc.get_sparse_core_info()
    nc, ns = info.num_cores, info.num_subcores
    nw = nc * ns
    ub = _BATCH // nw
    ib = _BATCH * _HIST // nw
    chunk = 128
    nchunk = ib // chunk
    nbuf = 5
    ngrp = nchunk // nbuf

    mesh = plsc.VectorSubcoreMesh(core_axis_name="c", subcore_axis_name="s")

    @functools.partial(
        pl.kernel,
        out_type=(
            jax.ShapeDtypeStruct((_BATCH, _D), jnp.float32),
            jax.ShapeDtypeStruct((_BATCH * _HIST, _D), jnp.float32),
        ),
        mesh=mesh,
        compiler_params=pltpu.CompilerParams(use_tc_tiling_on_sc=False),
        scratch_types=[
            pltpu.VMEM((ub,), jnp.int32),
            pltpu.VMEM((ub, _D), jnp.float32),
            pltpu.VMEM((ib,), jnp.int32),
            [pltpu.VMEM((chunk, _D), jnp.float32) for _ in range(nbuf)],
            pltpu.SemaphoreType.DMA,
            [pltpu.SemaphoreType.DMA for _ in range(nbuf)],
            [pltpu.SemaphoreType.DMA for _ in range(nbuf)],
        ],
    )
    def emb(uid, iid, utab, itab, uout, iout,
            uidx, urows, iidx, bufs, usem, gsems, wsems):
        wid = lax.axis_index("s") * nc + lax.axis_index("c")
        ubase = wid * ub
        ibase = wid * ib

        pltpu.sync_copy(uid.at[pl.ds(ubase, ub)], uidx)
        pltpu.async_copy(utab.at[uidx], urows, usem)
        pltpu.sync_copy(iid.at[pl.ds(ibase, ib)], iidx)
        pltpu.make_async_copy(utab.at[uidx], urows, usem).wait()
        pltpu.async_copy(urows, uout.at[pl.ds(ubase, ub)], usem)

        def gather(c, b):
            pltpu.async_copy(
                itab.at[iidx.at[pl.ds(c * chunk, chunk)]], bufs[b], gsems[b])

        def wait_gather(c, b):
            pltpu.make_async_copy(
                itab.at[iidx.at[pl.ds(c * chunk, chunk)]], bufs[b],
                gsems[b]).wait()

        def put(c, b):
            pltpu.async_copy(
                bufs[b], iout.at[pl.ds(ibase + c * chunk, chunk)], wsems[b])

        def wait_put(c, b):
            pltpu.make_async_copy(
                bufs[b], iout.at[pl.ds(ibase + c * chunk, chunk)],
                wsems[b]).wait()

        for b in range(nbuf):
            gather(b, b)

        @pl.loop(0, ngrp - 1)
        def _grp(g):
            c0 = g * nbuf
            for b in range(nbuf):
                wait_gather(c0 + b, b)
                put(c0 + b, b)
                wait_put(c0 + b, b)
                gather(c0 + nbuf + b, b)

        c0 = (ngrp - 1) * nbuf
        for b in range(nbuf):
            wait_gather(c0 + b, b)
            put(c0 + b, b)
            wait_put(c0 + b, b)

        pltpu.make_async_copy(urows, uout.at[pl.ds(ubase, ub)], usem).wait()

    return emb


def kernel(user_id, items_ids, user_table, item_table):
    emb = _build()
    uid = user_id.astype(jnp.int32)
    iid = items_ids.reshape(-1).astype(jnp.int32)
    user_eb, item_flat = emb(uid, iid, user_table, item_table)
    return user_eb, item_flat.reshape(_BATCH, _HIST, _D)
